# baseline (speedup 1.0000x reference)
import jax
import jax.numpy as jnp
import numpy as np
from jax import lax
from jax.experimental import pallas as pl
from jax.experimental.pallas import tpu as pltpu
from jax.experimental.pallas import tpu_sc as plsc

B, L, D = 64, 8192, 64
LANES = 16
CH = L // LANES
NBKT = 256
NPASS = 4
VSTEP = L // LANES
WROWS = 128
NWIN = L // WROWS

MINI32 = np.int32(-2147483648)


def _srl(x, n):
    return lax.shift_right_logical(x, lax.full(x.shape, np.int32(n), np.int32))


def _sc_body(qt, qe, kt, ke, qt_o, qe_o, kt_o, ke_o,
             tf32, ka, va, kb, vb, cnt, gbuf, sem0, sem1):
    c = lax.axis_index("c")
    s = lax.axis_index("s")
    wid = s * 2 + c
    lane = lax.iota(jnp.int32, LANES)
    ones = lax.full((LANES,), np.int32(1), np.int32)
    gsems = (sem0, sem1)

    for (t_in, e_in, t_out, e_out) in ((qt, qe, qt_o, qe_o),
                                       (kt, ke, kt_o, ke_o)):
        for r in range(2):
            row = wid * 2 + r
            rbase = row * L

            pltpu.sync_copy(t_in.at[pl.ds(rbase, L)], tf32)

            def premap(i, carry):
                x = tf32[pl.ds(i * LANES, LANES)]
                bits = lax.bitcast_convert_type(x, jnp.int32)
                ka[pl.ds(i * LANES, LANES)] = jnp.where(bits < 0, ~bits,
                                                        bits ^ MINI32)
                return carry
            lax.fori_loop(0, VSTEP, premap, 0)

            bufs = ((ka, va, kb, vb), (kb, vb, ka, va),
                    (ka, va, kb, vb), (kb, vb, ka, va))
            for p in range(NPASS):
                src_k, src_v, dst_k, dst_v = bufs[p]
                shift = 8 * p
                last = p == NPASS - 1

                def clear(j, carry):
                    cnt[pl.ds(j * LANES, LANES)] = lax.full((LANES,), np.int32(0), np.int32)
                    return carry
                lax.fori_loop(0, NBKT, clear, 0)

                def hist(tt, carry):
                    idxv = lane * CH + tt
                    k = plsc.load_gather(src_k, [idxv])
                    d = _srl(k, shift) & 255
                    plsc.addupdate_scatter(cnt, [d * LANES + lane], ones)
                    return carry
                lax.fori_loop(0, CH, hist, 0)

                def scan(j, carry):
                    x = cnt[pl.ds(j * LANES, LANES)]
                    inc = plsc.cumsum(x)
                    cnt[pl.ds(j * LANES, LANES)] = inc - x + carry
                    return carry + jnp.sum(x, axis=0)
                lax.fori_loop(0, NBKT, scan, np.int32(0))

                def permute(tt, carry):
                    idxv = lane * CH + tt
                    k = plsc.load_gather(src_k, [idxv])
                    d = _srl(k, shift) & 255
                    cidx = d * LANES + lane
                    rk = plsc.load_gather(cnt, [cidx])
                    plsc.store_scatter(cnt, [cidx], rk + 1)
                    plsc.store_scatter(dst_k, [rk], k)
                    v = idxv if p == 0 else plsc.load_gather(src_v, [idxv])
                    if last:
                        v = v + rbase
                    plsc.store_scatter(dst_v, [rk], v)
                    return carry
                lax.fori_loop(0, CH, permute, 0)

            def unmap(i, carry):
                u = ka[pl.ds(i * LANES, LANES)]
                bits = jnp.where(u < 0, u ^ MINI32, ~u)
                tf32[pl.ds(i * LANES, LANES)] = lax.bitcast_convert_type(
                    bits, jnp.float32)
                return carry
            lax.fori_loop(0, VSTEP, unmap, 0)
            pltpu.sync_copy(tf32, t_out.at[pl.ds(rbase, L)])

            def g_start(w, b):
                pltpu.async_copy(e_in.at[va.at[pl.ds(w * WROWS, WROWS)]],
                                 gbuf.at[b], gsems[b])

            def g_wait(w, b):
                pltpu.make_async_copy(
                    e_in.at[va.at[pl.ds(w * WROWS, WROWS)]],
                    gbuf.at[b], gsems[b]).wait()

            g_start(0, 0)
            g_start(1, 1)

            def ring(it, carry):
                for b in range(2):
                    w = it * 2 + b
                    g_wait(w, b)
                    pltpu.sync_copy(gbuf.at[b],
                                    e_out.at[pl.ds(rbase + w * WROWS, WROWS)])
                    g_start(w + 2, b)
                return carry
            lax.fori_loop(0, NWIN // 2 - 1, ring, 0)
            for b in range(2):
                w = NWIN - 2 + b
                g_wait(w, b)
                pltpu.sync_copy(gbuf.at[b],
                                e_out.at[pl.ds(rbase + w * WROWS, WROWS)])


def kernel(query_time, query_embed, key_time, key_embed):
    qt = query_time.reshape(B * L)
    qe = query_embed.reshape(B * L, D)
    kt = key_time.reshape(B * L)
    ke = key_embed.reshape(B * L, D)
    mesh = plsc.VectorSubcoreMesh(core_axis_name="c", subcore_axis_name="s")
    f = pl.kernel(
        _sc_body,
        out_type=(
            jax.ShapeDtypeStruct((B * L,), jnp.float32),
            jax.ShapeDtypeStruct((B * L, D), jnp.float32),
            jax.ShapeDtypeStruct((B * L,), jnp.float32),
            jax.ShapeDtypeStruct((B * L, D), jnp.float32),
        ),
        mesh=mesh,
        compiler_params=pltpu.CompilerParams(needs_layout_passes=False,
                                             use_tc_tiling_on_sc=False),
        scratch_types=[
            pltpu.VMEM((L,), jnp.float32),
            pltpu.VMEM((L,), jnp.int32),
            pltpu.VMEM((L,), jnp.int32),
            pltpu.VMEM((L,), jnp.int32),
            pltpu.VMEM((L,), jnp.int32),
            pltpu.VMEM((NBKT * LANES,), jnp.int32),
            pltpu.VMEM((2, WROWS, D), jnp.float32),
            pltpu.SemaphoreType.DMA,
            pltpu.SemaphoreType.DMA,
        ],
    )
    qt_s, qe_s, kt_s, ke_s = f(qt, qe, kt, ke)
    return (qt_s.reshape(B, L), qe_s.reshape(B, L, D),
            kt_s.reshape(B, L), ke_s.reshape(B, L, D))


# speedup vs baseline: 1.1857x; 1.1857x over previous
import jax
import jax.numpy as jnp
import numpy as np
from jax import lax
from jax.experimental import pallas as pl
from jax.experimental.pallas import tpu as pltpu
from jax.experimental.pallas import tpu_sc as plsc

B, L, D = 64, 8192, 64
LANES = 16
CH = L // LANES
NBKT = 256
NPASS = 4
VSTEP = L // LANES
RPC = 32
DP = D // 2

MINI32 = np.int32(-2147483648)


def _srl(x, n):
    return lax.shift_right_logical(x, lax.full(x.shape, np.int32(n), np.int32))


def _sc_body(qt, qe, kt, ke, qt_o, qe_o, kt_o, ke_o,
             tf32, ka, vx, kb, cnt, is0,
             inb0, inb1, orow0, orow1, isem0, isem1, osem0, osem1):
    c = lax.axis_index("c")
    s = lax.axis_index("s")
    lane = lax.iota(jnp.int32, LANES)
    ones = lax.full((LANES,), np.int32(1), np.int32)
    inbs = (inb0, inb1)
    orows = (orow0, orow1)
    isems = (isem0, isem1)
    osems = (osem0, osem1)

    for g, (t_in, e_in, t_out, e_out) in enumerate(
            ((qt, qe, qt_o, qe_o), (kt, ke, kt_o, ke_o))):
        for r in range(2):
            row = c * RPC + s * 2 + r
            slot = is0

            pltpu.sync_copy(t_in.at[row], tf32)

            def premap(i, carry):
                x = tf32[pl.ds(i * LANES, LANES)]
                bits = lax.bitcast_convert_type(x, jnp.int32)
                ka[pl.ds(i * LANES, LANES)] = jnp.where(bits < 0, ~bits,
                                                        bits ^ MINI32)
                return carry
            lax.fori_loop(0, VSTEP, premap, 0)

            bufs = ((ka, None, kb, vx), (kb, vx, ka, slot),
                    (ka, slot, kb, vx), (kb, vx, ka, slot))
            for p in range(NPASS):
                src_k, src_v, dst_k, dst_v = bufs[p]
                shift = 8 * p

                def clear(j, carry):
                    cnt[pl.ds(j * LANES, LANES)] = lax.full(
                        (LANES,), np.int32(0), np.int32)
                    return carry
                lax.fori_loop(0, NBKT, clear, 0)

                def hist(tt, carry):
                    idxv = lane * CH + tt
                    k = plsc.load_gather(src_k, [idxv])
                    d = _srl(k, shift) & 255
                    plsc.addupdate_scatter(cnt, [d * LANES + lane], ones)
                    return carry
                lax.fori_loop(0, CH, hist, 0)

                def scan(j, carry):
                    x = cnt[pl.ds(j * LANES, LANES)]
                    inc = plsc.cumsum(x)
                    cnt[pl.ds(j * LANES, LANES)] = inc - x + carry
                    return carry + jnp.sum(x, axis=0)
                lax.fori_loop(0, NBKT, scan, np.int32(0))

                def permute(tt, carry):
                    idxv = lane * CH + tt
                    k = plsc.load_gather(src_k, [idxv])
                    d = _srl(k, shift) & 255
                    cidx = d * LANES + lane
                    rk = plsc.load_gather(cnt, [cidx])
                    plsc.store_scatter(cnt, [cidx], rk + 1)
                    plsc.store_scatter(dst_k, [rk], k)
                    v = idxv if p == 0 else plsc.load_gather(src_v, [idxv])
                    plsc.store_scatter(dst_v, [rk], v)
                    return carry
                lax.fori_loop(0, CH, permute, 0)

            def unmap(i, carry):
                u = ka[pl.ds(i * LANES, LANES)]
                bits = jnp.where(u < 0, u ^ MINI32, ~u)
                tf32[pl.ds(i * LANES, LANES)] = lax.bitcast_convert_type(
                    bits, jnp.float32)
                return carry
            lax.fori_loop(0, VSTEP, unmap, 0)
            pltpu.sync_copy(tf32, t_out.at[row])

            def in_start(dp, b):
                pltpu.async_copy(e_in.at[row, pl.ds(dp * 2, 2)],
                                 inbs[b], isems[b])

            def in_wait(dp, b):
                pltpu.make_async_copy(e_in.at[row, pl.ds(dp * 2, 2)],
                                      inbs[b], isems[b]).wait()

            def out_start(dp, b):
                pltpu.async_copy(orows[b], e_out.at[row, pl.ds(dp * 2, 2)],
                                 osems[b])

            def out_wait(dp, b):
                pltpu.make_async_copy(orows[b],
                                      e_out.at[row, pl.ds(dp * 2, 2)],
                                      osems[b]).wait()

            def permute_pair(dp, b):
                in_wait(dp, b)

                def step(t, carry):
                    iv = slot[pl.ds(t * LANES, LANES)]
                    svec = lane + t * LANES
                    for r2 in range(2):
                        r2v = lax.full((LANES,), np.int32(r2), np.int32)
                        v = plsc.load_gather(inbs[b], [r2v, iv])
                        plsc.store_scatter(orows[b], [r2v, svec], v)
                    return carry
                lax.fori_loop(0, VSTEP, step, 0)

            in_start(0, 0)
            in_start(1, 1)
            for dp in range(2):
                permute_pair(dp, dp)
                out_start(dp, dp)
                in_start(dp + 2, dp)

            def ring(it, carry):
                for b in range(2):
                    dp = it * 2 + b
                    out_wait(dp - 2, b)
                    permute_pair(dp, b)
                    out_start(dp, b)
                    in_start(dp + 2, b)
                return carry
            lax.fori_loop(1, DP // 2 - 1, ring, 0)
            for b in range(2):
                dp = DP - 2 + b
                out_wait(dp - 2, b)
                permute_pair(dp, b)
                out_start(dp, b)
            for b in range(2):
                out_wait(DP - 2 + b, b)


def kernel(query_time, query_embed, key_time, key_embed):
    qe = query_embed.transpose(0, 2, 1)
    ke = key_embed.transpose(0, 2, 1)
    mesh = plsc.VectorSubcoreMesh(core_axis_name="c", subcore_axis_name="s")
    f = pl.kernel(
        _sc_body,
        out_type=(
            jax.ShapeDtypeStruct((B, L), jnp.float32),
            jax.ShapeDtypeStruct((B, D, L), jnp.float32),
            jax.ShapeDtypeStruct((B, L), jnp.float32),
            jax.ShapeDtypeStruct((B, D, L), jnp.float32),
        ),
        mesh=mesh,
        compiler_params=pltpu.CompilerParams(needs_layout_passes=False),
        scratch_types=[
            pltpu.VMEM((L,), jnp.float32),
            pltpu.VMEM((L,), jnp.int32),
            pltpu.VMEM((L,), jnp.int32),
            pltpu.VMEM((L,), jnp.int32),
            pltpu.VMEM((NBKT * LANES,), jnp.int32),
            pltpu.VMEM((L,), jnp.int32),
            pltpu.VMEM((2, L), jnp.float32),
            pltpu.VMEM((2, L), jnp.float32),
            pltpu.VMEM((2, L), jnp.float32),
            pltpu.VMEM((2, L), jnp.float32),
            pltpu.SemaphoreType.DMA,
            pltpu.SemaphoreType.DMA,
            pltpu.SemaphoreType.DMA,
            pltpu.SemaphoreType.DMA,
        ],
    )
    qt_s, qe_s, kt_s, ke_s = f(query_time, qe, key_time, ke)
    return (qt_s, qe_s.transpose(0, 2, 1),
            kt_s, ke_s.transpose(0, 2, 1))
